# Initial kernel scaffold; baseline (speedup 1.0000x reference)
import jax
import jax.numpy as jnp
from jax.experimental import pallas as pl


def kernel(features, weight_r, edge_index, edge_type):
    raise NotImplementedError("write your pallas kernel here")


# speedup vs baseline: 10.1737x; 10.1737x over previous
import dataclasses
import functools

import jax
import jax.numpy as jnp
from jax import lax
from jax.experimental import pallas as pl
from jax.experimental.pallas import tpu as pltpu
from jax.experimental.pallas import tpu_sc as plsc

N = 10000
E = 320000
F = 128
R = 16

NC = 2
NS = 16
NW = NC * NS
K = 80
NSUP = 5
G = 25
CPW = NSUP * G
ZCHUNKS = N // K
NWRITE = 10
WROWS = N // NWRITE
NB = 400
NBLK = N // NB
HROWS = 8
NVEC = N // 16

C_SELF = 0.2


def _hr_body(x_ref, w_ref, o_ref):
    x = x_ref[...]
    for r in range(R):
        o_ref[r] = jnp.dot(x, w_ref[r], preferred_element_type=jnp.float32)


def _compute_hr(features, weight_r):
    return pl.pallas_call(
        _hr_body,
        grid=(NBLK,),
        in_specs=[
            pl.BlockSpec((NB, F), lambda i: (i, 0)),
            pl.BlockSpec((R, F, F), lambda i: (0, 0, 0)),
        ],
        out_specs=pl.BlockSpec((R, NB, F), lambda i: (0, i, 0)),
        out_shape=jax.ShapeDtypeStruct((R, N, F), jnp.float32),
    )(features, weight_r)


GK = G * K


def _sc_degree(dst_deg, rowbase, lanes):
    mesh = plsc.VectorSubcoreMesh(core_axis_name="c", subcore_axis_name="s")
    cp = pltpu.CompilerParams()
    if "needs_layout_passes" in pltpu.CompilerParams.__dataclass_fields__:
        cp = dataclasses.replace(cp, needs_layout_passes=False)

    @functools.partial(
        pl.kernel,
        out_type=jax.ShapeDtypeStruct((NW, 1, N), jnp.float32),
        mesh=mesh,
        compiler_params=cp,
        scratch_types=[
            pltpu.VMEM((GK,), jnp.int32),
            pltpu.VMEM((HROWS * N,), jnp.float32),
            pltpu.VMEM((N,), jnp.float32),
            pltpu.VMEM((16,), jnp.int32),
            pltpu.VMEM((16,), jnp.int32),
        ],
    )
    def body(dst_hbm, rb_hbm, ln_hbm, d_out, dst_v, hist_v, red_v, rb_v, ln_v):
        cid = lax.axis_index("c")
        sid = lax.axis_index("s")
        wid = sid * NC + cid

        pltpu.sync_copy(rb_hbm, rb_v)
        pltpu.sync_copy(ln_hbm, ln_v)

        zeros = jnp.zeros((16,), jnp.float32)

        @pl.loop(0, NVEC)
        def _(j):
            for r in range(HROWS):
                hist_v[pl.ds(r * N + j * 16, 16)] = zeros

        rowbase16 = rb_v[...]
        lanes16 = ln_v[...]
        lo = lanes16 < HROWS
        hi = lanes16 >= HROWS
        ones16 = jnp.full((16,), 1.0, jnp.float32)

        @pl.loop(0, NSUP)
        def _(g):
            pltpu.sync_copy(dst_hbm.at[wid, g, 0], dst_v)

            @pl.loop(0, GK // 16)
            def _(j):
                d16 = dst_v[pl.ds(j * 16, 16)]
                flat = rowbase16 + d16
                plsc.addupdate_scatter(hist_v, [flat], ones16, mask=lo)
                plsc.addupdate_scatter(hist_v, [flat], ones16, mask=hi)

        @pl.loop(0, NVEC)
        def _(j):
            acc = hist_v[pl.ds(j * 16, 16)]
            for r in range(1, HROWS):
                acc = acc + hist_v[pl.ds(r * N + j * 16, 16)]
            red_v[pl.ds(j * 16, 16)] = acc

        pltpu.sync_copy(red_v, d_out.at[wid, 0])

    return body(dst_deg, rowbase, lanes)


def _sc_aggregate(hr_flat, src4, dst4, et4):
    mesh = plsc.VectorSubcoreMesh(core_axis_name="c", subcore_axis_name="s")

    @functools.partial(
        pl.kernel,
        out_type=[jax.ShapeDtypeStruct((NC, N, F), jnp.float32)],
        mesh=mesh,
        scratch_types=[
            pltpu.VMEM((G, K), jnp.int32),
            pltpu.VMEM((G, K), jnp.int32),
            pltpu.VMEM((G, K), jnp.int32),
            pltpu.VMEM((K, F), jnp.float32),
            pltpu.VMEM_SHARED((N, F), jnp.float32),
        ],
    )
    def body(hr_hbm, src_hbm, dst_hbm, et_hbm, m_out,
             src_v, dst_v, fidx_v, rows_v, m_sh):
        cid = lax.axis_index("c")
        sid = lax.axis_index("s")
        wid = sid * NC + cid

        @pl.loop(0, K)
        def _(i):
            @pl.loop(0, F // 16)
            def _(j):
                rows_v[i, pl.ds(j * 16, 16)] = jnp.zeros((16,), jnp.float32)

        @pl.loop(0, (ZCHUNKS + NS - 1) // NS)
        def _(t):
            z = sid + t * NS

            @pl.when(z < ZCHUNKS)
            def _():
                pltpu.sync_copy(rows_v, m_sh.at[pl.ds(z * K, K)])

        plsc.subcore_barrier()

        @pl.loop(0, NSUP)
        def _(g):
            pltpu.sync_copy(src_hbm.at[wid, g], src_v)
            pltpu.sync_copy(dst_hbm.at[wid, g], dst_v)
            pltpu.sync_copy(et_hbm.at[wid, g], fidx_v)

            @pl.loop(0, G)
            def _(r):
                @pl.loop(0, K // 16)
                def _(j):
                    sl = pl.ds(j * 16, 16)
                    fidx_v[r, sl] = fidx_v[r, sl] * N + src_v[r, sl]

            @pl.loop(0, G)
            def _(c):
                pltpu.sync_copy(hr_hbm.at[fidx_v.at[c]], rows_v)
                pltpu.sync_copy(rows_v, m_sh.at[dst_v.at[c]], add=True)

        plsc.subcore_barrier()

        @pl.when(sid < NWRITE)
        def _():
            off = sid * WROWS
            pltpu.sync_copy(m_sh.at[pl.ds(off, WROWS)],
                            m_out.at[cid, pl.ds(off, WROWS)])

    return body(hr_flat, src4, dst4, et4)[0]


def _final_body(x_ref, m_ref, d_ref, o_ref):
    x = x_ref[...]
    m = m_ref[0] + m_ref[1]
    deg_row = jnp.sum(d_ref[0], axis=0, keepdims=True)
    deg = deg_row.T
    deg_safe = jnp.maximum(deg, 1.0)
    h = jnp.maximum(C_SELF * x + (1.0 - C_SELF) * (m / deg_safe), 0.0)
    o_ref[...] = jnp.where(deg > 0.0, h, x)


def _combine(features, m_p, d_p):
    return pl.pallas_call(
        _final_body,
        grid=(NBLK,),
        in_specs=[
            pl.BlockSpec((NB, F), lambda i: (i, 0)),
            pl.BlockSpec((NC, NB, F), lambda i: (0, i, 0)),
            pl.BlockSpec((1, NW, NB), lambda i: (i, 0, 0)),
        ],
        out_specs=pl.BlockSpec((NB, F), lambda i: (i, 0)),
        out_shape=jax.ShapeDtypeStruct((N, F), jnp.float32),
    )(features, m_p, d_p)


def kernel(features, weight_r, edge_index, edge_type):
    src4 = edge_index[0].reshape(NW, NSUP, G, K)
    dst4 = edge_index[1].reshape(NW, NSUP, G, K)
    et4 = edge_type.reshape(NW, NSUP, G, K)
    dst_deg = edge_index[1].reshape(NW, NSUP, 1, GK)
    lanes = jnp.arange(16, dtype=jnp.int32)
    rowbase = (lanes & (HROWS - 1)) * N
    d_p = _sc_degree(dst_deg, rowbase, lanes)
    hr = _compute_hr(features, weight_r).reshape(R * N, F)
    m_p = _sc_aggregate(hr, src4, dst4, et4)
    d_pt = d_p.reshape(NW, NBLK, NB).transpose(1, 0, 2)
    return _combine(features, m_p, d_pt)


# speedup vs baseline: 12.0729x; 1.1867x over previous
import dataclasses
import functools

import jax
import jax.numpy as jnp
from jax import lax
from jax.experimental import pallas as pl
from jax.experimental.pallas import tpu as pltpu
from jax.experimental.pallas import tpu_sc as plsc

N = 10000
E = 320000
F = 128
R = 16

NC = 2
NS = 16
NW = NC * NS
K = 80
NSUP = 5
G = 25
CPW = NSUP * G
ZCHUNKS = N // K
NWRITE = 10
WROWS = N // NWRITE
NB = 400
NBLK = N // NB
HROWS = 8
NVEC = N // 16

C_SELF = 0.2


def _hr_body(x_ref, w_ref, o_ref):
    x = x_ref[...]
    for r in range(R):
        o_ref[r] = jnp.dot(x, w_ref[r], preferred_element_type=jnp.float32)


def _compute_hr(features, weight_r):
    return pl.pallas_call(
        _hr_body,
        grid=(NBLK,),
        in_specs=[
            pl.BlockSpec((NB, F), lambda i: (i, 0)),
            pl.BlockSpec((R, F, F), lambda i: (0, 0, 0)),
        ],
        out_specs=pl.BlockSpec((R, NB, F), lambda i: (0, i, 0)),
        out_shape=jax.ShapeDtypeStruct((R, N, F), jnp.float32),
    )(features, weight_r)


GK = G * K


def _sc_degree(dst_deg, rowbase, lanes):
    mesh = plsc.VectorSubcoreMesh(core_axis_name="c", subcore_axis_name="s")
    cp = pltpu.CompilerParams()
    if "needs_layout_passes" in pltpu.CompilerParams.__dataclass_fields__:
        cp = dataclasses.replace(cp, needs_layout_passes=False)

    @functools.partial(
        pl.kernel,
        out_type=jax.ShapeDtypeStruct((NW, 1, N), jnp.float32),
        mesh=mesh,
        compiler_params=cp,
        scratch_types=[
            pltpu.VMEM((GK,), jnp.int32),
            pltpu.VMEM((HROWS * N,), jnp.float32),
            pltpu.VMEM((N,), jnp.float32),
            pltpu.VMEM((16,), jnp.int32),
            pltpu.VMEM((16,), jnp.int32),
        ],
    )
    def body(dst_hbm, rb_hbm, ln_hbm, d_out, dst_v, hist_v, red_v, rb_v, ln_v):
        cid = lax.axis_index("c")
        sid = lax.axis_index("s")
        wid = sid * NC + cid

        pltpu.sync_copy(rb_hbm, rb_v)
        pltpu.sync_copy(ln_hbm, ln_v)

        zeros = jnp.zeros((16,), jnp.float32)

        @pl.loop(0, NVEC)
        def _(j):
            for r in range(HROWS):
                hist_v[pl.ds(r * N + j * 16, 16)] = zeros

        rowbase16 = rb_v[...]
        lanes16 = ln_v[...]
        lo = lanes16 < HROWS
        hi = lanes16 >= HROWS
        ones16 = jnp.full((16,), 1.0, jnp.float32)

        @pl.loop(0, NSUP)
        def _(g):
            pltpu.sync_copy(dst_hbm.at[wid, g, 0], dst_v)

            @pl.loop(0, GK // 16)
            def _(j):
                d16 = dst_v[pl.ds(j * 16, 16)]
                flat = rowbase16 + d16
                plsc.addupdate_scatter(hist_v, [flat], ones16, mask=lo)
                plsc.addupdate_scatter(hist_v, [flat], ones16, mask=hi)

        @pl.loop(0, NVEC)
        def _(j):
            acc = hist_v[pl.ds(j * 16, 16)]
            for r in range(1, HROWS):
                acc = acc + hist_v[pl.ds(r * N + j * 16, 16)]
            red_v[pl.ds(j * 16, 16)] = acc

        pltpu.sync_copy(red_v, d_out.at[wid, 0])

    return body(dst_deg, rowbase, lanes)


def _sc_aggregate(hr_flat, src4, dst4, et4):
    mesh = plsc.VectorSubcoreMesh(core_axis_name="c", subcore_axis_name="s")

    @functools.partial(
        pl.kernel,
        out_type=[jax.ShapeDtypeStruct((NC, N, F), jnp.float32)],
        mesh=mesh,
        scratch_types=[
            pltpu.VMEM((G, K), jnp.int32),
            pltpu.VMEM((G, K), jnp.int32),
            pltpu.VMEM((G, K), jnp.int32),
            pltpu.VMEM((K, F), jnp.float32),
            pltpu.VMEM((K, F), jnp.float32),
            pltpu.VMEM_SHARED((N, F), jnp.float32),
            pltpu.SemaphoreType.DMA,
            pltpu.SemaphoreType.DMA,
            pltpu.SemaphoreType.DMA,
            pltpu.SemaphoreType.DMA,
        ],
    )
    def body(hr_hbm, src_hbm, dst_hbm, et_hbm, m_out,
             src_v, dst_v, fidx_v, rows_a, rows_b, m_sh,
             sem_ga, sem_gb, sem_sa, sem_sb):
        cid = lax.axis_index("c")
        sid = lax.axis_index("s")
        wid = sid * NC + cid

        def fire_gather(c, buf, sem):
            pltpu.async_copy(hr_hbm.at[fidx_v.at[c]], buf, sem)

        def wait_gather(buf, sem):
            pltpu.make_async_copy(hr_hbm.at[pl.ds(0, K)], buf, sem).wait()

        def fire_scatter(c, buf, sem):
            pltpu.async_copy(buf, m_sh.at[dst_v.at[c]], sem, add=True)

        def wait_scatter(buf, sem):
            pltpu.make_async_copy(buf, m_sh.at[pl.ds(0, K)], sem).wait()

        @pl.loop(0, K)
        def _(i):
            @pl.loop(0, F // 16)
            def _(j):
                rows_a[i, pl.ds(j * 16, 16)] = jnp.zeros((16,), jnp.float32)

        @pl.loop(0, (ZCHUNKS + NS - 1) // NS)
        def _(t):
            z = sid + t * NS

            @pl.when(z < ZCHUNKS)
            def _():
                pltpu.sync_copy(rows_a, m_sh.at[pl.ds(z * K, K)])

        plsc.subcore_barrier()

        @pl.loop(0, NSUP)
        def _(g):
            pltpu.sync_copy(src_hbm.at[wid, g], src_v)
            pltpu.sync_copy(dst_hbm.at[wid, g], dst_v)
            pltpu.sync_copy(et_hbm.at[wid, g], fidx_v)

            @pl.loop(0, G)
            def _(r):
                @pl.loop(0, K // 16)
                def _(j):
                    sl = pl.ds(j * 16, 16)
                    fidx_v[r, sl] = fidx_v[r, sl] * N + src_v[r, sl]

            fire_gather(0, rows_a, sem_ga)

            @pl.loop(0, G // 2)
            def _(p):
                c0 = 2 * p
                wait_gather(rows_a, sem_ga)
                fire_gather(c0 + 1, rows_b, sem_gb)
                fire_scatter(c0, rows_a, sem_sa)
                wait_gather(rows_b, sem_gb)
                wait_scatter(rows_a, sem_sa)
                fire_gather(c0 + 2, rows_a, sem_ga)
                fire_scatter(c0 + 1, rows_b, sem_sb)
                wait_scatter(rows_b, sem_sb)

            wait_gather(rows_a, sem_ga)
            fire_scatter(G - 1, rows_a, sem_sa)
            wait_scatter(rows_a, sem_sa)

        plsc.subcore_barrier()

        @pl.when(sid < NWRITE)
        def _():
            off = sid * WROWS
            pltpu.sync_copy(m_sh.at[pl.ds(off, WROWS)],
                            m_out.at[cid, pl.ds(off, WROWS)])

    return body(hr_flat, src4, dst4, et4)[0]


def _final_body(x_ref, m_ref, d_ref, o_ref):
    x = x_ref[...]
    m = m_ref[0] + m_ref[1]
    deg_row = jnp.sum(d_ref[0], axis=0, keepdims=True)
    deg = deg_row.T
    deg_safe = jnp.maximum(deg, 1.0)
    h = jnp.maximum(C_SELF * x + (1.0 - C_SELF) * (m / deg_safe), 0.0)
    o_ref[...] = jnp.where(deg > 0.0, h, x)


def _combine(features, m_p, d_p):
    return pl.pallas_call(
        _final_body,
        grid=(NBLK,),
        in_specs=[
            pl.BlockSpec((NB, F), lambda i: (i, 0)),
            pl.BlockSpec((NC, NB, F), lambda i: (0, i, 0)),
            pl.BlockSpec((1, NW, NB), lambda i: (i, 0, 0)),
        ],
        out_specs=pl.BlockSpec((NB, F), lambda i: (i, 0)),
        out_shape=jax.ShapeDtypeStruct((N, F), jnp.float32),
    )(features, m_p, d_p)


def kernel(features, weight_r, edge_index, edge_type):
    src4 = edge_index[0].reshape(NW, NSUP, G, K)
    dst4 = edge_index[1].reshape(NW, NSUP, G, K)
    et4 = edge_type.reshape(NW, NSUP, G, K)
    dst_deg = edge_index[1].reshape(NW, NSUP, 1, GK)
    lanes = jnp.arange(16, dtype=jnp.int32)
    rowbase = (lanes & (HROWS - 1)) * N
    d_p = _sc_degree(dst_deg, rowbase, lanes)
    hr = _compute_hr(features, weight_r).reshape(R * N, F)
    m_p = _sc_aggregate(hr, src4, dst4, et4)
    d_pt = d_p.reshape(NW, NBLK, NB).transpose(1, 0, 2)
    return _combine(features, m_p, d_pt)
